# Initial kernel scaffold; baseline (speedup 1.0000x reference)
import jax
import jax.numpy as jnp
from jax.experimental import pallas as pl


def kernel(bonds, bond_atom_1, bond_atom_2, atoms, W1, b1, W2, b2, W3, b3):
    raise NotImplementedError("write your pallas kernel here")


# speedup vs baseline: 2.6524x; 2.6524x over previous
import functools

import jax
import jax.numpy as jnp
from jax import lax
from jax.experimental import pallas as pl
from jax.experimental.pallas import tpu as pltpu

try:
    from jax.experimental.pallas import tpu_sc as plsc
except ImportError:
    plsc = None

E = 1600000
N_ATOMS = 100000
ATOM_DIM = 32

_SLOPE = 11.0 / 48.0


def _make_sc_gather():
    info = plsc.get_sparse_core_info()
    nw = info.num_cores * info.num_subcores
    ew = E // nw
    C = 1000
    iters = ew // C
    assert ew % C == 0 and (ew % 8 == 0) and (C % 8 == 0)

    mesh = plsc.VectorSubcoreMesh(core_axis_name="c", subcore_axis_name="s")

    @functools.partial(
        pl.kernel,
        mesh=mesh,
        out_type=(
            jax.ShapeDtypeStruct((E, ATOM_DIM), jnp.float32),
            jax.ShapeDtypeStruct((E, ATOM_DIM), jnp.float32),
        ),
        scratch_types=[
            pltpu.VMEM((C,), jnp.int32),
            pltpu.VMEM((C,), jnp.int32),
            pltpu.VMEM((C, ATOM_DIM), jnp.float32),
            pltpu.VMEM((C, ATOM_DIM), jnp.float32),
            pltpu.SemaphoreType.DMA,
            pltpu.SemaphoreType.DMA,
        ],
        compiler_params=pltpu.CompilerParams(use_tc_tiling_on_sc=False),
    )
    def gather_kernel(atoms_hbm, idx1_hbm, idx2_hbm, out1_hbm, out2_hbm,
                      idx1_v, idx2_v, rows1_v, rows2_v, sem1, sem2):
        wid = lax.axis_index("s") * info.num_cores + lax.axis_index("c")
        base = wid * ew

        def body(i, _):
            off = base + i * C
            pltpu.sync_copy(idx1_hbm.at[pl.ds(off, C)], idx1_v)
            pltpu.sync_copy(idx2_hbm.at[pl.ds(off, C)], idx2_v)
            cp1 = pltpu.async_copy(atoms_hbm.at[idx1_v], rows1_v, sem1)
            cp2 = pltpu.async_copy(atoms_hbm.at[idx2_v], rows2_v, sem2)
            cp1.wait()
            cp2.wait()
            pltpu.sync_copy(rows1_v, out1_hbm.at[pl.ds(off, C)])
            pltpu.sync_copy(rows2_v, out2_hbm.at[pl.ds(off, C)])
            return 0

        lax.fori_loop(0, iters, body, 0)

    return gather_kernel


def _mlp_body(a1_ref, a2_ref, b_ref, w1_ref, b1_ref, w2_ref, b2_ref,
              w3_ref, b3_ref, o_ref):
    h = jnp.concatenate([a1_ref[...], a2_ref[...], b_ref[...]], axis=1)
    h = jnp.dot(h, w1_ref[...], preferred_element_type=jnp.float32) + b1_ref[...]
    h = jnp.where(h >= 0, h, _SLOPE * h)
    h = jnp.dot(h, w2_ref[...], preferred_element_type=jnp.float32) + b2_ref[...]
    h = jnp.where(h >= 0, h, _SLOPE * h)
    o_ref[...] = jnp.dot(h, w3_ref[...], preferred_element_type=jnp.float32) + b3_ref[...]


def _mlp_call(a1, a2, bonds, W1, b1, W2, b2, W3, b3, blk):
    grid = (E // blk,)
    full = lambda i: (0, 0)
    row = lambda i: (i, 0)
    return pl.pallas_call(
        _mlp_body,
        grid=grid,
        in_specs=[
            pl.BlockSpec((blk, ATOM_DIM), row),
            pl.BlockSpec((blk, ATOM_DIM), row),
            pl.BlockSpec((blk, ATOM_DIM), row),
            pl.BlockSpec(W1.shape, full),
            pl.BlockSpec((1, 64), full),
            pl.BlockSpec(W2.shape, full),
            pl.BlockSpec((1, 64), full),
            pl.BlockSpec(W3.shape, full),
            pl.BlockSpec((1, 32), full),
        ],
        out_specs=pl.BlockSpec((blk, 32), row),
        out_shape=jax.ShapeDtypeStruct((E, 32), jnp.float32),
        compiler_params=pltpu.CompilerParams(
            dimension_semantics=("arbitrary",),
        ),
    )(a1, a2, bonds, W1, b1, W2, b2, W3, b3)


def kernel(bonds, bond_atom_1, bond_atom_2, atoms, W1, b1, W2, b2, W3, b3):
    gather = _make_sc_gather()
    a1, a2 = gather(atoms, bond_atom_1.astype(jnp.int32),
                    bond_atom_2.astype(jnp.int32))
    return _mlp_call(a1, a2, bonds, W1, b1.reshape(1, 64), W2,
                     b2.reshape(1, 64), W3, b3.reshape(1, 32), blk=8000)


# speedup vs baseline: 4.0050x; 1.5099x over previous
import functools

import jax
import jax.numpy as jnp
from jax import lax
from jax.experimental import pallas as pl
from jax.experimental.pallas import tpu as pltpu

try:
    from jax.experimental.pallas import tpu_sc as plsc
except ImportError:
    plsc = None

E = 1600000
N_ATOMS = 100000
ATOM_DIM = 32
BLK = 8000
Q = BLK // 4

_SLOPE = 11.0 / 48.0


def _make_sc_gather():
    info = plsc.get_sparse_core_info()
    nw = info.num_cores * info.num_subcores
    ew = E // nw
    C = 1000
    iters = ew // C
    assert ew % C == 0 and C % 8 == 0 and Q % C == 0

    mesh = plsc.VectorSubcoreMesh(core_axis_name="c", subcore_axis_name="s")

    @functools.partial(
        pl.kernel,
        mesh=mesh,
        out_type=(
            jax.ShapeDtypeStruct((E // 4, 4 * ATOM_DIM), jnp.float32),
            jax.ShapeDtypeStruct((E // 4, 4 * ATOM_DIM), jnp.float32),
        ),
        scratch_types=[
            pltpu.VMEM((C,), jnp.int32),
            pltpu.VMEM((C,), jnp.int32),
            pltpu.VMEM((C, ATOM_DIM), jnp.float32),
            pltpu.VMEM((C, ATOM_DIM), jnp.float32),
            pltpu.SemaphoreType.DMA,
            pltpu.SemaphoreType.DMA,
        ],
        compiler_params=pltpu.CompilerParams(use_tc_tiling_on_sc=False),
    )
    def gather_kernel(atoms_hbm, idx1_hbm, idx2_hbm, out1_hbm, out2_hbm,
                      idx1_v, idx2_v, rows1_v, rows2_v, sem1, sem2):
        wid = lax.axis_index("s") * info.num_cores + lax.axis_index("c")
        ubase = wid * (ew // C)

        def body(i, _):
            u = ubase + i
            off = u * C
            per_blk = BLK // C
            per_grp = Q // C
            i_blk = u // per_blk
            k = (u % per_blk) // per_grp
            r = (u % per_grp) * C
            row = i_blk * Q + r
            col = ATOM_DIM * k
            pltpu.sync_copy(idx1_hbm.at[pl.ds(off, C)], idx1_v)
            pltpu.sync_copy(idx2_hbm.at[pl.ds(off, C)], idx2_v)
            cp1 = pltpu.async_copy(atoms_hbm.at[idx1_v], rows1_v, sem1)
            cp2 = pltpu.async_copy(atoms_hbm.at[idx2_v], rows2_v, sem2)
            cp1.wait()
            cp2.wait()
            pltpu.sync_copy(rows1_v,
                            out1_hbm.at[pl.ds(row, C), pl.ds(col, ATOM_DIM)])
            pltpu.sync_copy(rows2_v,
                            out2_hbm.at[pl.ds(row, C), pl.ds(col, ATOM_DIM)])
            return 0

        lax.fori_loop(0, iters, body, 0)

    return gather_kernel


def _unpack(p):
    return jnp.concatenate(
        [p[:, k * ATOM_DIM:(k + 1) * ATOM_DIM] for k in range(4)], axis=0)


def _mlp_body(a1_ref, a2_ref, b_ref, w1_ref, b1_ref, w2_ref, b2_ref,
              w3_ref, b3_ref, o_ref):
    a1 = _unpack(a1_ref[...])
    a2 = _unpack(a2_ref[...])
    h = jnp.concatenate([a1, a2, b_ref[...]], axis=1)
    h = jnp.dot(h, w1_ref[...], preferred_element_type=jnp.float32) + b1_ref[...]
    h = jnp.where(h >= 0, h, _SLOPE * h)
    h = jnp.dot(h, w2_ref[...], preferred_element_type=jnp.float32) + b2_ref[...]
    h = jnp.where(h >= 0, h, _SLOPE * h)
    o_ref[...] = jnp.dot(h, w3_ref[...], preferred_element_type=jnp.float32) + b3_ref[...]


def _mlp_call(a1, a2, bonds, W1, b1, W2, b2, W3, b3):
    grid = (E // BLK,)
    full = lambda i: (0, 0)
    row = lambda i: (i, 0)
    return pl.pallas_call(
        _mlp_body,
        grid=grid,
        in_specs=[
            pl.BlockSpec((Q, 4 * ATOM_DIM), row),
            pl.BlockSpec((Q, 4 * ATOM_DIM), row),
            pl.BlockSpec((BLK, ATOM_DIM), row),
            pl.BlockSpec(W1.shape, full),
            pl.BlockSpec((1, 64), full),
            pl.BlockSpec(W2.shape, full),
            pl.BlockSpec((1, 64), full),
            pl.BlockSpec(W3.shape, full),
            pl.BlockSpec((1, 32), full),
        ],
        out_specs=pl.BlockSpec((BLK, 32), row),
        out_shape=jax.ShapeDtypeStruct((E, 32), jnp.float32),
        compiler_params=pltpu.CompilerParams(
            dimension_semantics=("arbitrary",),
        ),
    )(a1, a2, bonds, W1, b1, W2, b2, W3, b3)


def kernel(bonds, bond_atom_1, bond_atom_2, atoms, W1, b1, W2, b2, W3, b3):
    gather = _make_sc_gather()
    a1, a2 = gather(atoms, bond_atom_1.astype(jnp.int32),
                    bond_atom_2.astype(jnp.int32))
    return _mlp_call(a1, a2, bonds, W1, b1.reshape(1, 64), W2,
                     b2.reshape(1, 64), W3, b3.reshape(1, 32))
